# Initial kernel scaffold; baseline (speedup 1.0000x reference)
import functools

import jax
import jax.numpy as jnp
from jax import lax
from jax.experimental import pallas as pl
from jax.experimental.pallas import tpu as pltpu
from jax.experimental.pallas import tpu_sc as plsc

NC = 2
NS = 16
L = 16
NW = NC * NS
DIM = 128
NJ = DIM // L
EPS2 = 1e-24


@functools.partial(jax.jit, static_argnames=("B",))
def _transh_sc(h_idx, r_idx, t_idx, ent_embs, rel_d_embs, rel_w_embs, *, B):
    per_w = B // NW
    CH = 128
    n_chunks = per_w // CH

    mesh = plsc.VectorSubcoreMesh(
        core_axis_name="c", subcore_axis_name="s", num_cores=NC, num_subcores=NS
    )
    out_type = (
        jax.ShapeDtypeStruct((B, DIM), jnp.float32),
        jax.ShapeDtypeStruct((B, DIM), jnp.float32),
        jax.ShapeDtypeStruct((B, DIM), jnp.float32),
    )

    @functools.partial(
        pl.kernel,
        out_type=out_type,
        mesh=mesh,
        scratch_types=[
            pltpu.VMEM((CH,), jnp.int32),
            pltpu.VMEM((CH,), jnp.int32),
            pltpu.VMEM((CH,), jnp.int32),
            pltpu.VMEM((CH, DIM), jnp.float32),
            pltpu.VMEM((CH, DIM), jnp.float32),
            pltpu.VMEM((CH, DIM), jnp.float32),
            pltpu.VMEM((CH, DIM), jnp.float32),
            pltpu.SemaphoreType.DMA,
        ],
    )
    def k(hi_hbm, ri_hbm, ti_hbm, ent_hbm, reld_hbm, relw_hbm,
          ho_hbm, ro_hbm, to_hbm,
          hi_v, ri_v, ti_v, h_v, d_v, t_v, w_v, sem):
        wid = lax.axis_index("s") * NC + lax.axis_index("c")
        for c in range(n_chunks):
            base = wid * per_w + c * CH
            pltpu.sync_copy(hi_hbm.at[pl.ds(base, CH)], hi_v)
            pltpu.sync_copy(ri_hbm.at[pl.ds(base, CH)], ri_v)
            pltpu.sync_copy(ti_hbm.at[pl.ds(base, CH)], ti_v)
            cp_h = pltpu.async_copy(ent_hbm.at[hi_v], h_v, sem)
            cp_d = pltpu.async_copy(reld_hbm.at[ri_v], d_v, sem)
            cp_t = pltpu.async_copy(ent_hbm.at[ti_v], t_v, sem)
            cp_w = pltpu.async_copy(relw_hbm.at[ri_v], w_v, sem)
            cp_h.wait()
            cp_d.wait()
            cp_t.wait()
            cp_w.wait()

            def row(i, _):
                wj = [w_v[i, pl.ds(j * L, L)] for j in range(NJ)]
                hj = [h_v[i, pl.ds(j * L, L)] for j in range(NJ)]
                dj = [d_v[i, pl.ds(j * L, L)] for j in range(NJ)]
                tj = [t_v[i, pl.ds(j * L, L)] for j in range(NJ)]
                a_ww = wj[0] * wj[0]
                a_hw = hj[0] * wj[0]
                a_dw = dj[0] * wj[0]
                a_tw = tj[0] * wj[0]
                for j in range(1, NJ):
                    a_ww = a_ww + wj[j] * wj[j]
                    a_hw = a_hw + hj[j] * wj[j]
                    a_dw = a_dw + dj[j] * wj[j]
                    a_tw = a_tw + tj[j] * wj[j]
                inv = 1.0 / jnp.maximum(jnp.sum(a_ww), EPS2)
                c_h = jnp.sum(a_hw) * inv
                c_d = jnp.sum(a_dw) * inv
                c_t = jnp.sum(a_tw) * inv
                for j in range(NJ):
                    h_v[i, pl.ds(j * L, L)] = hj[j] - c_h * wj[j]
                    d_v[i, pl.ds(j * L, L)] = dj[j] - c_d * wj[j]
                    t_v[i, pl.ds(j * L, L)] = tj[j] - c_t * wj[j]
                return 0

            lax.fori_loop(0, CH, row, 0)
            pltpu.sync_copy(h_v, ho_hbm.at[pl.ds(base, CH)])
            pltpu.sync_copy(d_v, ro_hbm.at[pl.ds(base, CH)])
            pltpu.sync_copy(t_v, to_hbm.at[pl.ds(base, CH)])

    return k(h_idx, r_idx, t_idx, ent_embs, rel_d_embs, rel_w_embs)


def kernel(triplets, ent_embs, rel_d_embs, rel_w_embs):
    B = triplets.shape[0]
    trip = triplets.astype(jnp.int32)
    h_idx = trip[:, 0]
    r_idx = trip[:, 1]
    t_idx = trip[:, 2]
    return _transh_sc(h_idx, r_idx, t_idx,
                      ent_embs.astype(jnp.float32),
                      rel_d_embs.astype(jnp.float32),
                      rel_w_embs.astype(jnp.float32), B=B)


# speedup vs baseline: 1.7592x; 1.7592x over previous
import functools

import jax
import jax.numpy as jnp
from jax import lax
from jax.experimental import pallas as pl
from jax.experimental.pallas import tpu as pltpu
from jax.experimental.pallas import tpu_sc as plsc

NC = 2
NS = 16
L = 16
NW = NC * NS
DIM = 128
NJ = DIM // L
EPS2 = 1e-24


@functools.partial(jax.jit, static_argnames=("B",))
def _transh_sc(h_idx, r_idx, t_idx, ent_embs, rel_d_embs, rel_w_embs, *, B):
    per_w = B // NW
    CH = 128
    n_chunks = per_w // CH

    mesh = plsc.VectorSubcoreMesh(
        core_axis_name="c", subcore_axis_name="s", num_cores=NC, num_subcores=NS
    )
    out_type = (
        jax.ShapeDtypeStruct((B, DIM), jnp.float32),
        jax.ShapeDtypeStruct((B, DIM), jnp.float32),
        jax.ShapeDtypeStruct((B, DIM), jnp.float32),
    )

    @functools.partial(
        pl.kernel,
        out_type=out_type,
        mesh=mesh,
        compiler_params=pltpu.CompilerParams(needs_layout_passes=False),
        scratch_types=[
            pltpu.VMEM((CH,), jnp.int32),
            pltpu.VMEM((CH,), jnp.int32),
            pltpu.VMEM((CH,), jnp.int32),
            pltpu.VMEM((CH, DIM), jnp.float32),
            pltpu.VMEM((CH, DIM), jnp.float32),
            pltpu.VMEM((CH, DIM), jnp.float32),
            pltpu.VMEM((CH, DIM), jnp.float32),
            pltpu.SemaphoreType.DMA,
        ],
    )
    def k(hi_hbm, ri_hbm, ti_hbm, ent_hbm, reld_hbm, relw_hbm,
          ho_hbm, ro_hbm, to_hbm,
          hi_v, ri_v, ti_v, h_v, d_v, t_v, w_v, sem):
        wid = lax.axis_index("s") * NC + lax.axis_index("c")
        for c in range(n_chunks):
            base = wid * per_w + c * CH
            pltpu.sync_copy(hi_hbm.at[pl.ds(base, CH)], hi_v)
            pltpu.sync_copy(ri_hbm.at[pl.ds(base, CH)], ri_v)
            pltpu.sync_copy(ti_hbm.at[pl.ds(base, CH)], ti_v)
            cp_h = pltpu.async_copy(ent_hbm.at[hi_v], h_v, sem)
            cp_d = pltpu.async_copy(reld_hbm.at[ri_v], d_v, sem)
            cp_t = pltpu.async_copy(ent_hbm.at[ti_v], t_v, sem)
            cp_w = pltpu.async_copy(relw_hbm.at[ri_v], w_v, sem)
            cp_h.wait()
            cp_d.wait()
            cp_t.wait()
            cp_w.wait()

            def row(i, _):
                wj = [w_v[i, pl.ds(j * L, L)] for j in range(NJ)]
                hj = [h_v[i, pl.ds(j * L, L)] for j in range(NJ)]
                dj = [d_v[i, pl.ds(j * L, L)] for j in range(NJ)]
                tj = [t_v[i, pl.ds(j * L, L)] for j in range(NJ)]
                a_ww = wj[0] * wj[0]
                a_hw = hj[0] * wj[0]
                a_dw = dj[0] * wj[0]
                a_tw = tj[0] * wj[0]
                for j in range(1, NJ):
                    a_ww = a_ww + wj[j] * wj[j]
                    a_hw = a_hw + hj[j] * wj[j]
                    a_dw = a_dw + dj[j] * wj[j]
                    a_tw = a_tw + tj[j] * wj[j]
                ww = jnp.broadcast_to(jnp.sum(a_ww), (L,))
                inv = 1.0 / jnp.maximum(ww, EPS2)
                c_h = jnp.broadcast_to(jnp.sum(a_hw), (L,)) * inv
                c_d = jnp.broadcast_to(jnp.sum(a_dw), (L,)) * inv
                c_t = jnp.broadcast_to(jnp.sum(a_tw), (L,)) * inv
                for j in range(NJ):
                    h_v[i, pl.ds(j * L, L)] = hj[j] - c_h * wj[j]
                    d_v[i, pl.ds(j * L, L)] = dj[j] - c_d * wj[j]
                    t_v[i, pl.ds(j * L, L)] = tj[j] - c_t * wj[j]
                return 0

            lax.fori_loop(0, CH, row, 0)
            pltpu.sync_copy(h_v, ho_hbm.at[pl.ds(base, CH)])
            pltpu.sync_copy(d_v, ro_hbm.at[pl.ds(base, CH)])
            pltpu.sync_copy(t_v, to_hbm.at[pl.ds(base, CH)])

    return k(h_idx, r_idx, t_idx, ent_embs, rel_d_embs, rel_w_embs)


def kernel(triplets, ent_embs, rel_d_embs, rel_w_embs):
    B = triplets.shape[0]
    trip = triplets.astype(jnp.int32)
    h_idx = trip[:, 0]
    r_idx = trip[:, 1]
    t_idx = trip[:, 2]
    return _transh_sc(h_idx, r_idx, t_idx,
                      ent_embs.astype(jnp.float32),
                      rel_d_embs.astype(jnp.float32),
                      rel_w_embs.astype(jnp.float32), B=B)


# speedup vs baseline: 2.2068x; 1.2545x over previous
import functools

import jax
import jax.numpy as jnp
from jax import lax
from jax.experimental import pallas as pl
from jax.experimental.pallas import tpu as pltpu
from jax.experimental.pallas import tpu_sc as plsc

NC = 2
NS = 16
L = 16
NW = NC * NS
DIM = 128
NJ = DIM // L
EPS2 = 1e-24
CH = 64


@functools.partial(jax.jit, static_argnames=("B",))
def _transh_sc(h_idx, r_idx, t_idx, ent_embs, rel_d_embs, rel_w_embs, *, B):
    per_w = B // NW
    n_chunks = per_w // CH

    mesh = plsc.VectorSubcoreMesh(
        core_axis_name="c", subcore_axis_name="s", num_cores=NC, num_subcores=NS
    )
    out_type = (
        jax.ShapeDtypeStruct((B, DIM), jnp.float32),
        jax.ShapeDtypeStruct((B, DIM), jnp.float32),
        jax.ShapeDtypeStruct((B, DIM), jnp.float32),
    )
    row_buf = pltpu.VMEM((CH, DIM), jnp.float32)

    @functools.partial(
        pl.kernel,
        out_type=out_type,
        mesh=mesh,
        compiler_params=pltpu.CompilerParams(needs_layout_passes=False),
        scratch_types=[
            pltpu.VMEM((n_chunks, CH), jnp.int32),
            pltpu.VMEM((n_chunks, CH), jnp.int32),
            pltpu.VMEM((n_chunks, CH), jnp.int32),
            [row_buf] * 4,
            [row_buf] * 4,
            [pltpu.SemaphoreType.DMA] * 2,
            [pltpu.SemaphoreType.DMA] * 2,
        ],
    )
    def k(hi_hbm, ri_hbm, ti_hbm, ent_hbm, reld_hbm, relw_hbm,
          ho_hbm, ro_hbm, to_hbm,
          hi_v, ri_v, ti_v, buf0, buf1, gsems, wsems):
        wid = lax.axis_index("s") * NC + lax.axis_index("c")
        base_row = wid * n_chunks
        pltpu.sync_copy(hi_hbm.at[pl.ds(base_row, n_chunks)], hi_v)
        pltpu.sync_copy(ri_hbm.at[pl.ds(base_row, n_chunks)], ri_v)
        pltpu.sync_copy(ti_hbm.at[pl.ds(base_row, n_chunks)], ti_v)
        bufs = (buf0, buf1)

        def fire_gathers(c, p):
            h_v, d_v, t_v, w_v = bufs[p]
            pltpu.async_copy(ent_hbm.at[hi_v.at[c]], h_v, gsems[p])
            pltpu.async_copy(reld_hbm.at[ri_v.at[c]], d_v, gsems[p])
            pltpu.async_copy(ent_hbm.at[ti_v.at[c]], t_v, gsems[p])
            pltpu.async_copy(relw_hbm.at[ri_v.at[c]], w_v, gsems[p])

        def wait_gathers(p):
            h_v, d_v, t_v, w_v = bufs[p]
            pltpu.make_async_copy(ent_hbm.at[hi_v.at[0]], h_v, gsems[p]).wait()
            pltpu.make_async_copy(reld_hbm.at[ri_v.at[0]], d_v, gsems[p]).wait()
            pltpu.make_async_copy(ent_hbm.at[ti_v.at[0]], t_v, gsems[p]).wait()
            pltpu.make_async_copy(relw_hbm.at[ri_v.at[0]], w_v, gsems[p]).wait()

        def fire_writeback(c, p):
            h_v, d_v, t_v, _ = bufs[p]
            base = wid * per_w + c * CH
            pltpu.async_copy(h_v, ho_hbm.at[pl.ds(base, CH)], wsems[p])
            pltpu.async_copy(d_v, ro_hbm.at[pl.ds(base, CH)], wsems[p])
            pltpu.async_copy(t_v, to_hbm.at[pl.ds(base, CH)], wsems[p])

        def wait_writeback(p):
            h_v, d_v, t_v, _ = bufs[p]
            pltpu.make_async_copy(h_v, ho_hbm.at[pl.ds(0, CH)], wsems[p]).wait()
            pltpu.make_async_copy(d_v, ro_hbm.at[pl.ds(0, CH)], wsems[p]).wait()
            pltpu.make_async_copy(t_v, to_hbm.at[pl.ds(0, CH)], wsems[p]).wait()

        def compute(p):
            h_v, d_v, t_v, w_v = bufs[p]

            def row(i, _):
                wj = [w_v[i, pl.ds(j * L, L)] for j in range(NJ)]
                hj = [h_v[i, pl.ds(j * L, L)] for j in range(NJ)]
                dj = [d_v[i, pl.ds(j * L, L)] for j in range(NJ)]
                tj = [t_v[i, pl.ds(j * L, L)] for j in range(NJ)]
                a_ww = wj[0] * wj[0]
                a_hw = hj[0] * wj[0]
                a_dw = dj[0] * wj[0]
                a_tw = tj[0] * wj[0]
                for j in range(1, NJ):
                    a_ww = a_ww + wj[j] * wj[j]
                    a_hw = a_hw + hj[j] * wj[j]
                    a_dw = a_dw + dj[j] * wj[j]
                    a_tw = a_tw + tj[j] * wj[j]
                ww = jnp.broadcast_to(jnp.sum(a_ww), (L,))
                inv = 1.0 / jnp.maximum(ww, EPS2)
                c_h = jnp.broadcast_to(jnp.sum(a_hw), (L,)) * inv
                c_d = jnp.broadcast_to(jnp.sum(a_dw), (L,)) * inv
                c_t = jnp.broadcast_to(jnp.sum(a_tw), (L,)) * inv
                for j in range(NJ):
                    h_v[i, pl.ds(j * L, L)] = hj[j] - c_h * wj[j]
                    d_v[i, pl.ds(j * L, L)] = dj[j] - c_d * wj[j]
                    t_v[i, pl.ds(j * L, L)] = tj[j] - c_t * wj[j]
                return 0

            lax.fori_loop(0, CH, row, 0)

        fire_gathers(0, 0)
        for c in range(n_chunks):
            p = c & 1
            if c + 1 < n_chunks:
                if c >= 1:
                    wait_writeback(1 - p)
                fire_gathers(c + 1, 1 - p)
            wait_gathers(p)
            compute(p)
            fire_writeback(c, p)
        wait_writeback((n_chunks - 1) & 1)
        if n_chunks > 1:
            wait_writeback(n_chunks & 1)

    hi2 = h_idx.reshape(B // CH, CH)
    ri2 = r_idx.reshape(B // CH, CH)
    ti2 = t_idx.reshape(B // CH, CH)
    return k(hi2, ri2, ti2, ent_embs, rel_d_embs, rel_w_embs)


def kernel(triplets, ent_embs, rel_d_embs, rel_w_embs):
    B = triplets.shape[0]
    trip = triplets.astype(jnp.int32)
    h_idx = trip[:, 0]
    r_idx = trip[:, 1]
    t_idx = trip[:, 2]
    return _transh_sc(h_idx, r_idx, t_idx,
                      ent_embs.astype(jnp.float32),
                      rel_d_embs.astype(jnp.float32),
                      rel_w_embs.astype(jnp.float32), B=B)
